# baseline (speedup 1.0000x reference)
import functools

import jax
import jax.numpy as jnp
from jax import lax
from jax.experimental import pallas as pl
from jax.experimental.pallas import tpu as pltpu
from jax.experimental.pallas import tpu_sc as plsc

L = 16
NC = 2
NS = 16
NW = NC * NS
P = 128


def _build_sc_interp(B, C, H, W, N):
    pts_total = B * N
    assert pts_total % NW == 0
    ppw = pts_total // NW
    assert ppw % P == 0
    n_chunks = ppw // P
    assert N % ppw == 0
    wpb = N // ppw
    assert P % L == 0

    mesh = plsc.VectorSubcoreMesh(core_axis_name="c", subcore_axis_name="s",
                                  num_cores=NC, num_subcores=NS)

    @functools.partial(
        pl.kernel,
        out_type=jax.ShapeDtypeStruct((B * C * N,), jnp.float32),
        mesh=mesh,
        scratch_types=[
            pltpu.VMEM((P,), jnp.float32),
            pltpu.VMEM((P,), jnp.float32),
            pltpu.VMEM((P,), jnp.float32),
            pltpu.VMEM((P,), jnp.float32),
            [[pltpu.VMEM((P,), jnp.int32) for _ in range(4)]
             for _ in range(4)],
            [[pltpu.VMEM((P,), jnp.float32) for _ in range(4)]
             for _ in range(4)],
            pltpu.VMEM((4 * P,), jnp.float32),
            pltpu.SemaphoreType.DMA,
        ],
    )
    def sc_interp(table_hbm, xy_hbm, out_hbm,
                  xs_v, ys_v, wx_v, wy_v, idx_v, g_v, out_v, sem):
        cid = lax.axis_index("c")
        sid = lax.axis_index("s")
        wid = sid * NC + cid
        b = wid // wpb
        n_base = (wid % wpb) * ppw
        HW = H * W

        def chunk_body(chunk, carry):
            n0 = n_base + chunk * P
            pltpu.sync_copy(xy_hbm.at[pl.ds(b * 2 * N + n0, P)], xs_v)
            pltpu.sync_copy(xy_hbm.at[pl.ds(b * 2 * N + N + n0, P)], ys_v)

            for g in range(P // L):
                sl = pl.ds(g * L, L)
                x = xs_v[sl] * float(W - 1)
                y = ys_v[sl] * float(H - 1)
                x0 = jnp.clip(x.astype(jnp.int32), 0, W - 2)
                y0 = jnp.clip(y.astype(jnp.int32), 0, H - 2)
                wx_v[sl] = x - x0.astype(jnp.float32)
                wy_v[sl] = y - y0.astype(jnp.float32)
                base = (b * C * H + y0) * W + x0
                for c in range(C):
                    fc = base + c * HW
                    idx_v[0][c][sl] = fc
                    idx_v[1][c][sl] = fc + 1
                    idx_v[2][c][sl] = fc + W
                    idx_v[3][c][sl] = fc + W + 1

            copies = []
            for k in range(4):
                for c in range(C):
                    copies.append(pltpu.async_copy(
                        table_hbm.at[idx_v[k][c]], g_v[k][c], sem))
            for cp in copies:
                cp.wait()

            for g in range(P // L):
                sl = pl.ds(g * L, L)
                wx = wx_v[sl]
                wy = wy_v[sl]
                ex = 1.0 - wx
                ey = 1.0 - wy
                w00 = ex * ey
                w01 = wx * ey
                w10 = ex * wy
                w11 = wx * wy
                for c in range(C):
                    out_v[pl.ds(c * P + g * L, L)] = (
                        g_v[0][c][sl] * w00 + g_v[1][c][sl] * w01
                        + g_v[2][c][sl] * w10 + g_v[3][c][sl] * w11)

            for c in range(C):
                pltpu.sync_copy(
                    out_v.at[pl.ds(c * P, P)],
                    out_hbm.at[pl.ds((b * C + c) * N + n0, P)])
            return carry

        lax.fori_loop(0, n_chunks, chunk_body, 0)

    return sc_interp


@jax.jit
def kernel(R, XY_pc):
    B, C, H, W = R.shape
    N = XY_pc.shape[-1]
    table = R.reshape(B * C * H * W)
    xy = XY_pc.reshape(B * 2 * N)
    sc_interp = _build_sc_interp(B, C, H, W, N)
    out = sc_interp(table, xy)
    return out.reshape(B, C, N)
